# baseline (speedup 1.0000x reference)
import functools

import jax
import jax.numpy as jnp
from jax import lax
from jax.experimental import pallas as pl
from jax.experimental.pallas import tpu as pltpu
from jax.experimental.pallas import tpu_sc as plsc

F = 26
B = 16384
V = 100000
D = 16

NC = 2
NS = 16
NW = NC * NS
CHUNK = B // NW
IPD = 128
NJ = CHUNK // IPD
L = 16


def _body(idx_hbm, tab_hbm, out_hbm, idxv, sidxv, pbuf, rows, gsem, ssem):
    c = lax.axis_index("c")
    s = lax.axis_index("s")
    wid = c * NS + s
    b0 = wid * CHUNK

    iota = lax.iota(jnp.int32, L)
    for j in range(NJ):
        for l in range(IPD // L):
            sl = pl.ds(l * L, L)
            pbuf[j, sl] = iota * F + (b0 + j * IPD + l * L) * F

    def seg(f, carry):
        r0 = f * (B // IPD) + wid * NJ
        pltpu.sync_copy(idx_hbm.at[pl.ds(r0, NJ)], idxv)
        off = f * V
        for j in range(NJ):
            for l in range(IPD // L):
                sl = pl.ds(l * L, L)
                idxv[j, sl] = idxv[j, sl] + off
                sidxv[j, sl] = pbuf[j, sl] + f
        gc = [
            pltpu.async_copy(
                tab_hbm.at[idxv.at[j]], rows.at[pl.ds(j * IPD, IPD)], gsem
            )
            for j in range(NJ)
        ]
        for cpy in gc:
            cpy.wait()
        sc = [
            pltpu.async_copy(
                rows.at[pl.ds(j * IPD, IPD)], out_hbm.at[sidxv.at[j]], ssem
            )
            for j in range(NJ)
        ]
        for cpy in sc:
            cpy.wait()
        return carry

    lax.fori_loop(0, F, seg, 0)


def kernel(indices, tables):
    idx2d = indices.reshape(F * B // IPD, IPD)
    tab_flat = tables.reshape(F * V, D)
    mesh = plsc.VectorSubcoreMesh(core_axis_name="c", subcore_axis_name="s")
    run = functools.partial(
        pl.kernel,
        out_type=jax.ShapeDtypeStruct((B * F, D), jnp.float32),
        mesh=mesh,
        compiler_params=pltpu.CompilerParams(use_tc_tiling_on_sc=False),
        scratch_types=[
            pltpu.VMEM((NJ, IPD), jnp.int32),
            pltpu.VMEM((NJ, IPD), jnp.int32),
            pltpu.VMEM((NJ, IPD), jnp.int32),
            pltpu.VMEM((CHUNK, D), jnp.float32),
            pltpu.SemaphoreType.DMA,
            pltpu.SemaphoreType.DMA,
        ],
    )(_body)
    out = run(idx2d, tab_flat)
    return out.reshape(B, F * D)


# speedup vs baseline: 4.9208x; 4.9208x over previous
import functools

import jax
import jax.numpy as jnp
from jax import lax
from jax.experimental import pallas as pl
from jax.experimental.pallas import tpu as pltpu
from jax.experimental.pallas import tpu_sc as plsc

F = 26
B = 16384
V = 100000
D = 16

NC = 2
NS = 16
NW = NC * NS
NT = F * D
TPW = NT // NW
BC = 8192
L = 16


def _body(idx_hbm, tab_hbm, out_hbm, row_v, idx_v, out_v):
    c = lax.axis_index("c")
    s = lax.axis_index("s")
    wid = s * NC + c
    t0 = wid * TPW

    def task(k, carry):
        t = t0 + k
        f = lax.shift_right_logical(t, 4)
        d = lax.bitwise_and(t, D - 1)
        pltpu.sync_copy(tab_hbm.at[f, d], row_v)
        for cb in range(B // BC):
            b0 = cb * BC
            pltpu.sync_copy(idx_hbm.at[f, pl.ds(b0, BC)], idx_v)

            def gat(i, c2):
                sl = pl.ds(i * L, L)
                out_v[sl] = plsc.load_gather(row_v, [idx_v[sl]])
                return c2

            lax.fori_loop(0, BC // L, gat, 0, unroll=8)
            pltpu.sync_copy(out_v, out_hbm.at[t, pl.ds(b0, BC)])
        return carry

    lax.fori_loop(0, TPW, task, 0)


def kernel(indices, tables):
    tab_t = jnp.transpose(tables, (0, 2, 1))
    mesh = plsc.VectorSubcoreMesh(core_axis_name="c", subcore_axis_name="s")
    run = functools.partial(
        pl.kernel,
        out_type=jax.ShapeDtypeStruct((NT, B), jnp.float32),
        mesh=mesh,
        compiler_params=pltpu.CompilerParams(needs_layout_passes=False),
        scratch_types=[
            pltpu.VMEM((V,), jnp.float32),
            pltpu.VMEM((BC,), jnp.int32),
            pltpu.VMEM((BC,), jnp.float32),
        ],
    )(_body)
    out_t = run(indices, tab_t)
    return jnp.transpose(out_t, (1, 0)).reshape(B, F * D)


# speedup vs baseline: 8.3797x; 1.7029x over previous
import functools

import jax
import jax.numpy as jnp
from jax import lax
from jax.experimental import pallas as pl
from jax.experimental.pallas import tpu as pltpu
from jax.experimental.pallas import tpu_sc as plsc

F = 26
B = 16384
V = 100000
D = 16

NC = 2
NS = 16
NW = NC * NS
NT = F * D
TPW = NT // NW
BC = 8192
L = 16


def _body(idx_hbm, tab_hbm, out_hbm, row_v, idx_v, out_v):
    c = lax.axis_index("c")
    s = lax.axis_index("s")
    wid = s * NC + c
    t0 = wid * TPW

    def task(k, carry):
        t = t0 + k
        f = lax.shift_right_logical(t, 4)
        d = lax.bitwise_and(t, D - 1)
        pltpu.sync_copy(tab_hbm.at[f, d], row_v)
        for cb in range(B // BC):
            b0 = cb * BC
            pltpu.sync_copy(idx_hbm.at[f, pl.ds(b0, BC)], idx_v)

            @plsc.parallel_loop(0, BC, L, unroll=8)
            def gat(i):
                sl = pl.ds(i, L)
                out_v[sl] = plsc.load_gather(row_v, [idx_v[sl]])
            pltpu.sync_copy(out_v, out_hbm.at[t, pl.ds(b0, BC)])
        return carry

    lax.fori_loop(0, TPW, task, 0)


def kernel(indices, tables):
    tab_t = jnp.transpose(tables, (0, 2, 1))
    mesh = plsc.VectorSubcoreMesh(core_axis_name="c", subcore_axis_name="s")
    run = functools.partial(
        pl.kernel,
        out_type=jax.ShapeDtypeStruct((NT, B), jnp.float32),
        mesh=mesh,
        compiler_params=pltpu.CompilerParams(needs_layout_passes=False),
        scratch_types=[
            pltpu.VMEM((V,), jnp.float32),
            pltpu.VMEM((BC,), jnp.int32),
            pltpu.VMEM((BC,), jnp.float32),
        ],
    )(_body)
    out_t = run(indices, tab_t)
    return jnp.transpose(out_t, (1, 0)).reshape(B, F * D)


# speedup vs baseline: 9.6389x; 1.1503x over previous
import functools

import jax
import jax.numpy as jnp
from jax import lax
from jax.experimental import pallas as pl
from jax.experimental.pallas import tpu as pltpu
from jax.experimental.pallas import tpu_sc as plsc

F = 26
B = 16384
V = 100000
D = 16

NC = 2
NS = 16
NW = NC * NS
NT = F * D
TPW = NT // NW
BC = 4096
NCB = B // BC
L = 16


def _body(idx_hbm, tab_hbm, out_hbm, row_v, idx2, out2,
          rsem, isem0, isem1, osem0, osem1):
    c = lax.axis_index("c")
    s = lax.axis_index("s")
    wid = s * NC + c
    t0 = wid * TPW
    isems = (isem0, isem1)
    osems = (osem0, osem1)

    def task(k, carry):
        t = t0 + k
        f = lax.shift_right_logical(t, 4)
        d = lax.bitwise_and(t, D - 1)
        rcp = pltpu.async_copy(tab_hbm.at[f, d], row_v, rsem)
        icps = [
            pltpu.async_copy(idx_hbm.at[f, pl.ds(0, BC)], idx2.at[0], isems[0]),
            pltpu.async_copy(idx_hbm.at[f, pl.ds(BC, BC)], idx2.at[1], isems[1]),
        ]

        @pl.when(k > 0)
        def _drain_prev():
            for pb in range(2):
                pltpu.make_async_copy(
                    out2.at[pb], out_hbm.at[0, pl.ds(pb * BC, BC)], osems[pb]
                ).wait()

        rcp.wait()
        ocps = [None, None]
        for cb in range(NCB):
            pb = cb & 1
            icps[cb].wait()
            if ocps[pb] is not None:
                ocps[pb].wait()

            @plsc.parallel_loop(0, BC, L, unroll=8)
            def gat(i):
                sl = pl.ds(i, L)
                out2[pb, sl] = plsc.load_gather(row_v, [idx2[pb, sl]])

            if cb + 2 < NCB:
                icps.append(
                    pltpu.async_copy(
                        idx_hbm.at[f, pl.ds((cb + 2) * BC, BC)],
                        idx2.at[pb], isems[pb],
                    )
                )
            ocps[pb] = pltpu.async_copy(
                out2.at[pb], out_hbm.at[t, pl.ds(cb * BC, BC)], osems[pb]
            )
        return carry

    lax.fori_loop(0, TPW, task, 0)
    for pb in range(2):
        pltpu.make_async_copy(
            out2.at[pb], out_hbm.at[0, pl.ds(pb * BC, BC)], osems[pb]
        ).wait()


def kernel(indices, tables):
    tab_t = jnp.transpose(tables, (0, 2, 1))
    mesh = plsc.VectorSubcoreMesh(core_axis_name="c", subcore_axis_name="s")
    run = functools.partial(
        pl.kernel,
        out_type=jax.ShapeDtypeStruct((NT, B), jnp.float32),
        mesh=mesh,
        compiler_params=pltpu.CompilerParams(needs_layout_passes=False),
        scratch_types=[
            pltpu.VMEM((V,), jnp.float32),
            pltpu.VMEM((2, BC), jnp.int32),
            pltpu.VMEM((2, BC), jnp.float32),
            pltpu.SemaphoreType.DMA,
            pltpu.SemaphoreType.DMA,
            pltpu.SemaphoreType.DMA,
            pltpu.SemaphoreType.DMA,
            pltpu.SemaphoreType.DMA,
        ],
    )(_body)
    out_t = run(indices, tab_t)
    return jnp.transpose(out_t, (1, 0)).reshape(B, F * D)


# speedup vs baseline: 10.3675x; 1.0756x over previous
import functools

import jax
import jax.numpy as jnp
from jax import lax
from jax.experimental import pallas as pl
from jax.experimental.pallas import tpu as pltpu
from jax.experimental.pallas import tpu_sc as plsc

F = 26
B = 16384
V = 100000
D = 16

NC = 2
NS = 16
NW = NC * NS
NT = F * D
TPW = NT // NW
BC = 4096
NCB = B // BC
L = 16


def _body(idx_hbm, tab_hbm, out_hbm, row_v, idx_f, out2,
          rsem, osem0, osem1):
    c = lax.axis_index("c")
    s = lax.axis_index("s")
    wid = s * NC + c
    t0 = wid * TPW
    osems = (osem0, osem1)

    def task(k, carry):
        t = t0 + k
        f = lax.shift_right_logical(t, 4)
        d = lax.bitwise_and(t, D - 1)
        rcp = pltpu.async_copy(tab_hbm.at[f, d], row_v, rsem)

        @pl.when(k > 0)
        def _drain_prev():
            for pb in range(2):
                pltpu.make_async_copy(
                    out2.at[pb], out_hbm.at[0, pl.ds(pb * BC, BC)], osems[pb]
                ).wait()

        @pl.when(jnp.logical_or(k == 0, d == 0))
        def _stage_indices():
            pltpu.sync_copy(idx_hbm.at[f], idx_f)

        rcp.wait()
        ocps = [None, None]
        for cb in range(NCB):
            pb = cb & 1
            if ocps[pb] is not None:
                ocps[pb].wait()

            @plsc.parallel_loop(0, BC, L, unroll=16)
            def gat(i):
                out2[pb, pl.ds(i, L)] = plsc.load_gather(
                    row_v, [idx_f[pl.ds(cb * BC + i, L)]]
                )

            ocps[pb] = pltpu.async_copy(
                out2.at[pb], out_hbm.at[t, pl.ds(cb * BC, BC)], osems[pb]
            )
        return carry

    lax.fori_loop(0, TPW, task, 0)
    for pb in range(2):
        pltpu.make_async_copy(
            out2.at[pb], out_hbm.at[0, pl.ds(pb * BC, BC)], osems[pb]
        ).wait()


def kernel(indices, tables):
    tab_t = jnp.transpose(tables, (0, 2, 1))
    mesh = plsc.VectorSubcoreMesh(core_axis_name="c", subcore_axis_name="s")
    run = functools.partial(
        pl.kernel,
        out_type=jax.ShapeDtypeStruct((NT, B), jnp.float32),
        mesh=mesh,
        compiler_params=pltpu.CompilerParams(needs_layout_passes=False),
        scratch_types=[
            pltpu.VMEM((V,), jnp.float32),
            pltpu.VMEM((B,), jnp.int32),
            pltpu.VMEM((2, BC), jnp.float32),
            pltpu.SemaphoreType.DMA,
            pltpu.SemaphoreType.DMA,
            pltpu.SemaphoreType.DMA,
        ],
    )(_body)
    out_t = run(indices, tab_t)
    return jnp.transpose(out_t, (1, 0)).reshape(B, F * D)
